# baseline (speedup 1.0000x reference)
import functools
import jax
import jax.numpy as jnp
from jax import lax
from jax.experimental import pallas as pl
from jax.experimental.pallas import tpu as pltpu, tpu_sc as plsc

MODEL_DIM = 1024
MAX_LEN = 32768
BATCH = 16384

_info = plsc.get_sparse_core_info()
_NC, _NS = _info.num_cores, _info.num_subcores
_NW = _NC * _NS
_BPW = BATCH // _NW
_CHUNK = 16
_NCHUNK = _BPW // _CHUNK
_NBUF = 3


@functools.partial(
    pl.kernel,
    mesh=plsc.VectorSubcoreMesh(core_axis_name="c", subcore_axis_name="s"),
    out_type=jax.ShapeDtypeStruct((BATCH, MODEL_DIM), jnp.float32),
    scratch_types=(
        [pltpu.VMEM((_BPW,), jnp.int32)]
        + [pltpu.VMEM((_CHUNK, MODEL_DIM), jnp.float32)] * _NBUF
        + [pltpu.VMEM_SHARED((_NS, _NBUF, _CHUNK, MODEL_DIM), jnp.float32)]
        + [pltpu.SemaphoreType.DMA] * (3 * _NBUF)
    ),
)
def _sc_gather(x_hbm, pe_hbm, out_hbm, idx_v, *rest):
    bufs = rest[:_NBUF]
    spmem = rest[_NBUF]
    g_sems = rest[_NBUF + 1:2 * _NBUF + 1]
    h1_sems = rest[2 * _NBUF + 1:3 * _NBUF + 1]
    h2_sems = rest[3 * _NBUF + 1:]

    wid = lax.axis_index("s") * _NC + lax.axis_index("c")
    sid = lax.axis_index("s")
    base = wid * _BPW
    pltpu.sync_copy(x_hbm.at[pl.ds(base, _BPW)], idx_v)

    def gather(c, slot):
        return pltpu.async_copy(
            pe_hbm.at[idx_v.at[pl.ds(c * _CHUNK, _CHUNK)]],
            bufs[slot], g_sems[slot],
        )

    def hop1(slot):
        return pltpu.async_copy(bufs[slot], spmem.at[sid, slot], h1_sems[slot])

    def hop2(c, slot):
        return pltpu.async_copy(
            spmem.at[sid, slot],
            out_hbm.at[pl.ds(base + c * _CHUNK, _CHUNK)], h2_sems[slot],
        )

    gathers = [None] * _NBUF
    h1s = [None] * _NBUF
    h2s = [None] * _NBUF
    for b in range(_NBUF - 1):
        gathers[b] = gather(b, b)
    for c in range(_NCHUNK):
        slot = c % _NBUF
        if c > 0:
            sp = (c - 1) % _NBUF
            h1s[sp].wait()
            h2s[sp] = hop2(c - 1, sp)
        pre = c + _NBUF - 1
        if pre < _NCHUNK:
            gathers[pre % _NBUF] = gather(pre, pre % _NBUF)
        gathers[slot].wait()
        if h2s[slot] is not None:
            h2s[slot].wait()
        h1s[slot] = hop1(slot)
    last = (_NCHUNK - 1) % _NBUF
    h1s[last].wait()
    h2s[last] = hop2(_NCHUNK - 1, last)
    for b in range(_NBUF):
        if h2s[b] is not None:
            h2s[b].wait()


def kernel(x, pe):
    return _sc_gather(x.astype(jnp.int32), pe)


# speedup vs baseline: 1.0158x; 1.0158x over previous
import functools
import jax
import jax.numpy as jnp
from jax import lax
from jax.experimental import pallas as pl
from jax.experimental.pallas import tpu as pltpu, tpu_sc as plsc

MODEL_DIM = 1024
MAX_LEN = 32768
BATCH = 16384

_info = plsc.get_sparse_core_info()
_NC, _NS = _info.num_cores, _info.num_subcores
_NW = _NC * _NS
_BPW = BATCH // _NW
_CHUNK = 32
_NCHUNK = _BPW // _CHUNK
_NBUF = 3


@functools.partial(
    pl.kernel,
    mesh=plsc.VectorSubcoreMesh(core_axis_name="c", subcore_axis_name="s"),
    out_type=jax.ShapeDtypeStruct((BATCH, MODEL_DIM), jnp.float32),
    scratch_types=(
        [pltpu.VMEM((_BPW,), jnp.int32)]
        + [pltpu.VMEM((_CHUNK, MODEL_DIM), jnp.float32)] * _NBUF
        + [pltpu.SemaphoreType.DMA] * (2 * _NBUF)
    ),
)
def _sc_gather(x_hbm, pe_hbm, out_hbm, idx_v, *bufs_and_sems):
    bufs = bufs_and_sems[:_NBUF]
    in_sems = bufs_and_sems[_NBUF:2 * _NBUF]
    out_sems = bufs_and_sems[2 * _NBUF:]

    wid = lax.axis_index("s") * _NC + lax.axis_index("c")
    base = wid * _BPW
    pltpu.sync_copy(x_hbm.at[pl.ds(base, _BPW)], idx_v)

    def gather(c, slot):
        return pltpu.async_copy(
            pe_hbm.at[idx_v.at[pl.ds(c * _CHUNK, _CHUNK)]],
            bufs[slot], in_sems[slot],
        )

    def put(c, slot):
        return pltpu.async_copy(
            bufs[slot], out_hbm.at[pl.ds(base + c * _CHUNK, _CHUNK)],
            out_sems[slot],
        )

    gathers = [None] * _NBUF
    puts = [None] * _NBUF
    for b in range(_NBUF - 1):
        gathers[b] = gather(b, b)
    for c in range(_NCHUNK):
        slot = c % _NBUF
        pre = c + _NBUF - 1
        if pre < _NCHUNK:
            s2 = pre % _NBUF
            if puts[s2] is not None:
                puts[s2].wait()
            gathers[s2] = gather(pre, s2)
        gathers[slot].wait()
        puts[slot] = put(c, slot)
    for b in range(_NBUF):
        if puts[b] is not None:
            puts[b].wait()


def kernel(x, pe):
    return _sc_gather(x.astype(jnp.int32), pe)
